# Initial kernel scaffold; baseline (speedup 1.0000x reference)
import jax
import jax.numpy as jnp
from jax.experimental import pallas as pl


def kernel(output, target):
    raise NotImplementedError("write your pallas kernel here")


# speedup vs baseline: 39.9494x; 39.9494x over previous
import functools

import jax
import jax.numpy as jnp
from jax.experimental import pallas as pl

NB, NA, NC, NH, NW = 32, 5, 13, 19, 19
NCELL = NH * NW
NANCH = NA * NCELL
NT = 50
E2M1 = 6.38905609893065
INV_E2M1 = 1.0 / E2M1
INV_E2M1E = 1.0 / (E2M1 + 1e-5)
SIL_THRESH = 0.6
OBJECT_SCALE = 5.0


def _region_loss_body(pred_ref, tgt_ref, gx_ref, gy_ref, out_ref):
    b = pl.program_id(0)
    t = tgt_ref[0]

    g1 = t[:, 1]
    zero_ind = (g1 == 0.0).astype(jnp.float32)
    row = jax.lax.broadcasted_iota(jnp.int32, (NT, NT), 0)
    col = jax.lax.broadcasted_iota(jnp.int32, (NT, NT), 1)
    tri = (col <= row).astype(jnp.float32)
    zcount = jnp.sum(tri * zero_ind[None, :], axis=1)
    valid = zcount == 0.0

    conf_sum = jnp.zeros((NT, NANCH), jnp.float32)
    for k in range(9):
        xraw = pred_ref[0, 2 * k, :]
        yraw = pred_ref[0, 2 * k + 1, :]
        if k == 0:
            xraw = jax.nn.sigmoid(xraw)
            yraw = jax.nn.sigmoid(yraw)
        px = (xraw + gx_ref[0]) * (640.0 / 19.0)
        py = (yraw + gy_ref[0]) * (480.0 / 19.0)
        gx = t[:, 1 + 2 * k] * 640.0
        gy = t[:, 2 + 2 * k] * 480.0
        dx = gx[:, None] - px[None, :]
        dy = gy[:, None] - py[None, :]
        dist = jnp.sqrt(dx * dx + dy * dy)
        c = jnp.where(dist < 80.0,
                      (jnp.exp(2.0 - dist * 0.025) - 1.0) * INV_E2M1, 0.0)
        conf_sum = conf_sum + c
    confs = jnp.where(valid[:, None], conf_sum * (1.0 / 9.0), 0.0)
    cur = jnp.max(confs, axis=0)
    m = (cur <= SIL_THRESH).astype(jnp.float32)
    confsig = jax.nn.sigmoid(pred_ref[0, 18, :])
    noobj = jnp.sum(m * confsig * confsig)

    gi = jnp.clip((g1 * 19.0).astype(jnp.int32), 0, NW - 1)
    gj = jnp.clip((t[:, 2] * 19.0).astype(jnp.int32), 0, NH - 1)
    cell = gj * NW + gi
    same = (cell[:, None] == cell[None, :]) & valid[None, :] & (col > row)
    later_dup = jnp.sum(same.astype(jnp.float32), axis=1) > 0.0
    winner = (valid & jnp.logical_not(later_dup)).astype(jnp.float32)

    lane = jax.lax.broadcasted_iota(jnp.int32, (NT, NCELL), 1)
    onehot = (lane == cell[:, None]).astype(jnp.float32)
    vals0 = pred_ref[0, :, 0:NCELL]
    cls = vals0[19:32]
    mx = jnp.max(cls, axis=0)
    lse = mx + jnp.log(jnp.sum(jnp.exp(cls - mx[None, :]), axis=0))
    ext = jnp.concatenate([vals0, m[None, 0:NCELL], lse[None, :]], axis=0)
    gathered = jax.lax.dot_general(
        onehot, ext, (((1,), (1,)), ((), ())),
        preferred_element_type=jnp.float32)

    gi_f = gi.astype(jnp.float32)
    gj_f = gj.astype(jnp.float32)
    coord = jnp.zeros((NT,), jnp.float32)
    cft_sum = jnp.zeros((NT,), jnp.float32)
    for k in range(9):
        xk = gathered[:, 2 * k]
        yk = gathered[:, 2 * k + 1]
        if k == 0:
            xk = jax.nn.sigmoid(xk)
            yk = jax.nn.sigmoid(yk)
        dxk = t[:, 1 + 2 * k] * 19.0 - gi_f - xk
        dyk = t[:, 2 + 2 * k] * 19.0 - gj_f - yk
        coord = coord + dxk * dxk + dyk * dyk
        sx = dxk * (640.0 / 19.0)
        sy = dyk * (480.0 / 19.0)
        distk = jnp.sqrt(sx * sx + sy * sy)
        ck = jnp.where(distk < 80.0,
                       (jnp.exp(2.0 - distk * 0.025) - 1.0) * INV_E2M1E, 0.0)
        cft_sum = cft_sum + ck
    cft = cft_sum * (1.0 / 9.0)

    confg = jax.nn.sigmoid(gathered[:, 18])
    m0g = gathered[:, 32]
    lseg = gathered[:, 33]
    label = jnp.clip(t[:, 0].astype(jnp.int32), 0, NC - 1)
    lbl_oh = (jax.lax.broadcasted_iota(jnp.int32, (NT, NC), 1)
              == label[:, None]).astype(jnp.float32)
    logit_lbl = jnp.sum(lbl_oh * gathered[:, 19:32], axis=1)

    box = (0.5 * coord
           + 0.5 * OBJECT_SCALE * (confg - cft) ** 2
           - 0.5 * m0g * confg * confg
           + (lseg - logit_lbl))
    partial = (0.5 * noobj + jnp.sum(winner * box)) * jnp.ones((1, 1), jnp.float32)

    @pl.when(b == 0)
    def _():
        out_ref[...] = partial

    @pl.when(b != 0)
    def _():
        out_ref[...] = out_ref[...] + partial


@functools.partial(jax.jit, static_argnames=())
def kernel(output, target):
    pred = output.reshape(NB, NA, 19 + NC, NCELL)
    pred = pred.transpose(0, 2, 1, 3).reshape(NB, 19 + NC, NANCH)
    tgt = target.reshape(NB, NT, 21)
    ii = jnp.tile(jnp.arange(NW, dtype=jnp.float32)[None, :], (NH, 1)).reshape(-1)
    jj = jnp.tile(jnp.arange(NH, dtype=jnp.float32)[:, None], (1, NW)).reshape(-1)
    gx = jnp.tile(ii, (NA,)).reshape(1, NANCH)
    gy = jnp.tile(jj, (NA,)).reshape(1, NANCH)

    res = pl.pallas_call(
        _region_loss_body,
        grid=(NB,),
        in_specs=[
            pl.BlockSpec((1, 19 + NC, NANCH), lambda b: (b, 0, 0)),
            pl.BlockSpec((1, NT, 21), lambda b: (b, 0, 0)),
            pl.BlockSpec((1, NANCH), lambda b: (0, 0)),
            pl.BlockSpec((1, NANCH), lambda b: (0, 0)),
        ],
        out_specs=pl.BlockSpec((1, 1), lambda b: (0, 0)),
        out_shape=jax.ShapeDtypeStruct((1, 1), jnp.float32),
    )(pred, tgt, gx, gy)
    return res[0, 0]


# speedup vs baseline: 61.9545x; 1.5508x over previous
import functools

import jax
import jax.numpy as jnp
from jax.experimental import pallas as pl

NB, NA, NC, NH, NW = 32, 5, 13, 19, 19
NCH = 19 + NC
NCELL = NH * NW
NT = 50
E2M1 = 6.38905609893065
INV9E2M1 = 1.0 / (9.0 * E2M1)
INV9E2M1E = 1.0 / (9.0 * (E2M1 + 1e-5))
SIL_THRESH = 0.6
OBJECT_SCALE = 5.0


def _region_loss_body(pred_ref, tgt_ref, gx_ref, gy_ref, out_ref):
    b = pl.program_id(0)
    t = tgt_ref[0]

    g1 = t[:, 1]
    zero_ind = (g1 == 0.0).astype(jnp.float32)
    row = jax.lax.broadcasted_iota(jnp.int32, (NT, NT), 0)
    col = jax.lax.broadcasted_iota(jnp.int32, (NT, NT), 1)
    tri = (col <= row).astype(jnp.float32)
    zcount = jnp.sum(tri * zero_ind[None, :], axis=1)
    valid = zcount == 0.0
    valid_f = valid.astype(jnp.float32)

    noobj = jnp.zeros((), jnp.float32)
    m0 = None
    for a in range(NA):
        base = a * NCH
        conf_sum = jnp.zeros((NT, NCELL), jnp.float32)
        for k in range(9):
            xraw = pred_ref[0, base + 2 * k, :]
            yraw = pred_ref[0, base + 2 * k + 1, :]
            if k == 0:
                xraw = jax.nn.sigmoid(xraw)
                yraw = jax.nn.sigmoid(yraw)
            px = (xraw + gx_ref[0]) * (16.0 / 19.0)
            py = (yraw + gy_ref[0]) * (12.0 / 19.0)
            gx = t[:, 1 + 2 * k] * 16.0
            gy = t[:, 2 + 2 * k] * 12.0
            dx = gx[:, None] - px[None, :]
            dy = gy[:, None] - py[None, :]
            d40 = jnp.sqrt(dx * dx + dy * dy)
            conf_sum = conf_sum + jnp.maximum(jnp.exp(2.0 - d40) - 1.0, 0.0)
        confs = conf_sum * valid_f[:, None]
        cur = jnp.max(confs, axis=0) * INV9E2M1
        m = (cur <= SIL_THRESH).astype(jnp.float32)
        confsig = jax.nn.sigmoid(pred_ref[0, base + 18, :])
        noobj = noobj + jnp.sum(m * confsig * confsig)
        if a == 0:
            m0 = m

    gi = jnp.clip((g1 * 19.0).astype(jnp.int32), 0, NW - 1)
    gj = jnp.clip((t[:, 2] * 19.0).astype(jnp.int32), 0, NH - 1)
    cell = gj * NW + gi
    same = (cell[:, None] == cell[None, :]) & valid[None, :] & (col > row)
    later_dup = jnp.sum(same.astype(jnp.float32), axis=1) > 0.0
    winner = (valid & jnp.logical_not(later_dup)).astype(jnp.float32)

    lane = jax.lax.broadcasted_iota(jnp.int32, (NT, NCELL), 1)
    onehot = (lane == cell[:, None]).astype(jnp.float32)
    vals0 = pred_ref[0, 0:NCH, :]
    cls = vals0[19:NCH]
    mx = jnp.max(cls, axis=0)
    lse = mx + jnp.log(jnp.sum(jnp.exp(cls - mx[None, :]), axis=0))
    ext = jnp.concatenate([vals0, m0[None, :], lse[None, :]], axis=0)
    gathered = jax.lax.dot_general(
        onehot, ext, (((1,), (1,)), ((), ())),
        preferred_element_type=jnp.float32)

    gi_f = gi.astype(jnp.float32)
    gj_f = gj.astype(jnp.float32)
    coord = jnp.zeros((NT,), jnp.float32)
    cft_sum = jnp.zeros((NT,), jnp.float32)
    for k in range(9):
        xk = gathered[:, 2 * k]
        yk = gathered[:, 2 * k + 1]
        if k == 0:
            xk = jax.nn.sigmoid(xk)
            yk = jax.nn.sigmoid(yk)
        dxk = t[:, 1 + 2 * k] * 19.0 - gi_f - xk
        dyk = t[:, 2 + 2 * k] * 19.0 - gj_f - yk
        coord = coord + dxk * dxk + dyk * dyk
        sx = dxk * (16.0 / 19.0)
        sy = dyk * (12.0 / 19.0)
        d40 = jnp.sqrt(sx * sx + sy * sy)
        cft_sum = cft_sum + jnp.maximum(jnp.exp(2.0 - d40) - 1.0, 0.0)
    cft = cft_sum * INV9E2M1E

    confg = jax.nn.sigmoid(gathered[:, 18])
    m0g = gathered[:, 32]
    lseg = gathered[:, 33]
    label = jnp.clip(t[:, 0].astype(jnp.int32), 0, NC - 1)
    lbl_oh = (jax.lax.broadcasted_iota(jnp.int32, (NT, NC), 1)
              == label[:, None]).astype(jnp.float32)
    logit_lbl = jnp.sum(lbl_oh * gathered[:, 19:NCH], axis=1)

    box = (0.5 * coord
           + 0.5 * OBJECT_SCALE * (confg - cft) ** 2
           - 0.5 * m0g * confg * confg
           + (lseg - logit_lbl))
    partial = (0.5 * noobj + jnp.sum(winner * box)) * jnp.ones((1, 1), jnp.float32)

    @pl.when(b == 0)
    def _():
        out_ref[...] = partial

    @pl.when(b != 0)
    def _():
        out_ref[...] = out_ref[...] + partial


@functools.partial(jax.jit, static_argnames=())
def kernel(output, target):
    pred = output.reshape(NB, NA * NCH, NCELL)
    tgt = target.reshape(NB, NT, 21)
    gx = jnp.tile(jnp.arange(NW, dtype=jnp.float32)[None, :],
                  (NH, 1)).reshape(1, NCELL)
    gy = jnp.tile(jnp.arange(NH, dtype=jnp.float32)[:, None],
                  (1, NW)).reshape(1, NCELL)

    res = pl.pallas_call(
        _region_loss_body,
        grid=(NB,),
        in_specs=[
            pl.BlockSpec((1, NA * NCH, NCELL), lambda b: (b, 0, 0)),
            pl.BlockSpec((1, NT, 21), lambda b: (b, 0, 0)),
            pl.BlockSpec((1, NCELL), lambda b: (0, 0)),
            pl.BlockSpec((1, NCELL), lambda b: (0, 0)),
        ],
        out_specs=pl.BlockSpec((1, 1), lambda b: (0, 0)),
        out_shape=jax.ShapeDtypeStruct((1, 1), jnp.float32),
    )(pred, tgt, gx, gy)
    return res[0, 0]


# speedup vs baseline: 72.1173x; 1.1640x over previous
import functools

import jax
import jax.numpy as jnp
from jax.experimental import pallas as pl

NB, NA, NC, NH, NW = 32, 5, 13, 19, 19
NCH = 19 + NC
NCELL = NH * NW
NT = 50
E2M1 = 6.38905609893065
INV9E2M1 = 1.0 / (9.0 * E2M1)
INV9E2M1E = 1.0 / (9.0 * (E2M1 + 1e-5))
SIL_THRESH = 0.6
OBJECT_SCALE = 5.0


def _region_loss_body(pred_ref, tgt_ref, gx_ref, gy_ref, out_ref):
    b = pl.program_id(0)
    t = tgt_ref[0]

    g1 = t[:, 1]
    zero_ind = (g1 == 0.0).astype(jnp.float32)
    row = jax.lax.broadcasted_iota(jnp.int32, (NT, NT), 0)
    col = jax.lax.broadcasted_iota(jnp.int32, (NT, NT), 1)
    tri = (col <= row).astype(jnp.float32)
    zcount = jnp.sum(tri * zero_ind[None, :], axis=1)
    valid = zcount == 0.0
    valid_f = valid.astype(jnp.float32)

    noobj = jnp.zeros((), jnp.float32)
    m0 = None
    for a in range(NA):
        base = a * NCH
        conf_sum = jnp.zeros((NT, NCELL), jnp.float32)
        for k in range(9):
            xraw = pred_ref[0, base + 2 * k, :]
            yraw = pred_ref[0, base + 2 * k + 1, :]
            if k == 0:
                xraw = jax.nn.sigmoid(xraw)
                yraw = jax.nn.sigmoid(yraw)
            px = (xraw + gx_ref[0]) * (16.0 / 19.0)
            py = (yraw + gy_ref[0]) * (12.0 / 19.0)
            gx = t[:, 1 + 2 * k] * 16.0
            gy = t[:, 2 + 2 * k] * 12.0
            dx = gx[:, None] - px[None, :]
            dy = gy[:, None] - py[None, :]
            s = dx * dx + dy * dy
            d40 = s * jax.lax.rsqrt(jnp.maximum(s, 1e-30))
            e = jnp.exp2(2.8853900817779268 - d40 * 1.4426950408889634)
            conf_sum = conf_sum + jnp.maximum(e - 1.0, 0.0)
        confs = conf_sum * valid_f[:, None]
        cur = jnp.max(confs, axis=0) * INV9E2M1
        m = (cur <= SIL_THRESH).astype(jnp.float32)
        confsig = jax.nn.sigmoid(pred_ref[0, base + 18, :])
        noobj = noobj + jnp.sum(m * confsig * confsig)
        if a == 0:
            m0 = m

    gi = jnp.clip((g1 * 19.0).astype(jnp.int32), 0, NW - 1)
    gj = jnp.clip((t[:, 2] * 19.0).astype(jnp.int32), 0, NH - 1)
    cell = gj * NW + gi
    same = (cell[:, None] == cell[None, :]) & valid[None, :] & (col > row)
    later_dup = jnp.sum(same.astype(jnp.float32), axis=1) > 0.0
    winner = (valid & jnp.logical_not(later_dup)).astype(jnp.float32)

    lane = jax.lax.broadcasted_iota(jnp.int32, (NT, NCELL), 1)
    onehot = (lane == cell[:, None]).astype(jnp.float32)
    vals0 = pred_ref[0, 0:NCH, :]
    cls = vals0[19:NCH]
    mx = jnp.max(cls, axis=0)
    lse = mx + jnp.log(jnp.sum(jnp.exp(cls - mx[None, :]), axis=0))
    ext = jnp.concatenate([vals0, m0[None, :], lse[None, :]], axis=0)
    gathered = jax.lax.dot_general(
        onehot, ext, (((1,), (1,)), ((), ())),
        preferred_element_type=jnp.float32)

    gi_f = gi.astype(jnp.float32)
    gj_f = gj.astype(jnp.float32)
    coord = jnp.zeros((NT,), jnp.float32)
    cft_sum = jnp.zeros((NT,), jnp.float32)
    for k in range(9):
        xk = gathered[:, 2 * k]
        yk = gathered[:, 2 * k + 1]
        if k == 0:
            xk = jax.nn.sigmoid(xk)
            yk = jax.nn.sigmoid(yk)
        dxk = t[:, 1 + 2 * k] * 19.0 - gi_f - xk
        dyk = t[:, 2 + 2 * k] * 19.0 - gj_f - yk
        coord = coord + dxk * dxk + dyk * dyk
        sx = dxk * (16.0 / 19.0)
        sy = dyk * (12.0 / 19.0)
        s = sx * sx + sy * sy
        d40 = s * jax.lax.rsqrt(jnp.maximum(s, 1e-30))
        cft_sum = cft_sum + jnp.maximum(jnp.exp(2.0 - d40) - 1.0, 0.0)
    cft = cft_sum * INV9E2M1E

    confg = jax.nn.sigmoid(gathered[:, 18])
    m0g = gathered[:, 32]
    lseg = gathered[:, 33]
    label = jnp.clip(t[:, 0].astype(jnp.int32), 0, NC - 1)
    lbl_oh = (jax.lax.broadcasted_iota(jnp.int32, (NT, NC), 1)
              == label[:, None]).astype(jnp.float32)
    logit_lbl = jnp.sum(lbl_oh * gathered[:, 19:NCH], axis=1)

    box = (0.5 * coord
           + 0.5 * OBJECT_SCALE * (confg - cft) ** 2
           - 0.5 * m0g * confg * confg
           + (lseg - logit_lbl))
    partial = (0.5 * noobj + jnp.sum(winner * box)) * jnp.ones((1, 1), jnp.float32)

    @pl.when(b == 0)
    def _():
        out_ref[...] = partial

    @pl.when(b != 0)
    def _():
        out_ref[...] = out_ref[...] + partial


@functools.partial(jax.jit, static_argnames=())
def kernel(output, target):
    pred = output.reshape(NB, NA * NCH, NCELL)
    tgt = target.reshape(NB, NT, 21)
    gx = jnp.tile(jnp.arange(NW, dtype=jnp.float32)[None, :],
                  (NH, 1)).reshape(1, NCELL)
    gy = jnp.tile(jnp.arange(NH, dtype=jnp.float32)[:, None],
                  (1, NW)).reshape(1, NCELL)

    res = pl.pallas_call(
        _region_loss_body,
        grid=(NB,),
        in_specs=[
            pl.BlockSpec((1, NA * NCH, NCELL), lambda b: (b, 0, 0)),
            pl.BlockSpec((1, NT, 21), lambda b: (b, 0, 0)),
            pl.BlockSpec((1, NCELL), lambda b: (0, 0)),
            pl.BlockSpec((1, NCELL), lambda b: (0, 0)),
        ],
        out_specs=pl.BlockSpec((1, 1), lambda b: (0, 0)),
        out_shape=jax.ShapeDtypeStruct((1, 1), jnp.float32),
    )(pred, tgt, gx, gy)
    return res[0, 0]
